# baseline (speedup 1.0000x reference)
import functools

import jax
import jax.numpy as jnp
from jax import lax
from jax.experimental import pallas as pl
from jax.experimental.pallas import tpu as pltpu
from jax.experimental.pallas import tpu_sc as plsc

NC = 2
NS = 16
NW = NC * NS
BATCH = 16384
RPW = BATCH // NW
FB = 64
MF_D = 10
ML_D = 32


VU = 1_000_000
VI = 100_000
CH = 128
NCH = RPW // CH


def _sc_gather(u1, i1, mfu_f, mfi_f, mlu_f, mli_f):
    mesh = plsc.VectorSubcoreMesh(core_axis_name="c", subcore_axis_name="s")

    @functools.partial(
        pl.kernel,
        mesh=mesh,
        compiler_params=pltpu.CompilerParams(use_tc_tiling_on_sc=False),
        out_type=(
            jax.ShapeDtypeStruct((MF_D, BATCH), jnp.float32),
            jax.ShapeDtypeStruct((MF_D, BATCH), jnp.float32),
            jax.ShapeDtypeStruct((ML_D, BATCH), jnp.float32),
            jax.ShapeDtypeStruct((ML_D, BATCH), jnp.float32),
        ),
        scratch_types=[
            pltpu.VMEM((RPW,), jnp.int32),
            pltpu.VMEM((RPW,), jnp.int32),
            pltpu.VMEM((MF_D, CH), jnp.int32),
            pltpu.VMEM((MF_D, CH), jnp.int32),
            pltpu.VMEM((ML_D, CH), jnp.int32),
            pltpu.VMEM((ML_D, CH), jnp.int32),
            pltpu.VMEM((MF_D, RPW), jnp.float32),
            pltpu.VMEM((MF_D, RPW), jnp.float32),
            pltpu.VMEM((ML_D, RPW), jnp.float32),
            pltpu.VMEM((ML_D, RPW), jnp.float32),
            pltpu.SemaphoreType.DMA,
        ],
    )
    def k(u_hbm, i_hbm, mfu_hbm, mfi_hbm, mlu_hbm, mli_hbm,
          omfu, omfi, omlu, omli,
          u_s, i_s, ofu, ofi, olu, oli, bmfu, bmfi, bmlu, bmli, sem):
        w = lax.axis_index("s") * NC + lax.axis_index("c")
        base = w * RPW
        pltpu.sync_copy(u_hbm.at[pl.ds(base, RPW)], u_s)
        pltpu.sync_copy(i_hbm.at[pl.ds(base, RPW)], i_s)

        def chunk(c, carry):
            for s in range(CH // 16):
                uvec = u_s[pl.ds(c * CH + 16 * s, 16)]
                ivec = i_s[pl.ds(c * CH + 16 * s, 16)]
                for d in range(MF_D):
                    ofu[d, 16 * s:16 * (s + 1)] = uvec + d * VU
                    ofi[d, 16 * s:16 * (s + 1)] = ivec + d * VI
                for d in range(ML_D):
                    olu[d, 16 * s:16 * (s + 1)] = uvec + d * VU
                    oli[d, 16 * s:16 * (s + 1)] = ivec + d * VI
            for d in range(MF_D):
                pltpu.async_copy(
                    mfu_hbm.at[ofu.at[d]],
                    bmfu.at[d, pl.ds(c * CH, CH)], sem)
                pltpu.async_copy(
                    mfi_hbm.at[ofi.at[d]],
                    bmfi.at[d, pl.ds(c * CH, CH)], sem)
            for d in range(ML_D):
                pltpu.async_copy(
                    mlu_hbm.at[olu.at[d]],
                    bmlu.at[d, pl.ds(c * CH, CH)], sem)
                pltpu.async_copy(
                    mli_hbm.at[oli.at[d]],
                    bmli.at[d, pl.ds(c * CH, CH)], sem)
            for d in range(MF_D):
                pltpu.make_async_copy(
                    mfu_hbm.at[ofu.at[d]],
                    bmfu.at[d, pl.ds(c * CH, CH)], sem).wait()
                pltpu.make_async_copy(
                    mfi_hbm.at[ofi.at[d]],
                    bmfi.at[d, pl.ds(c * CH, CH)], sem).wait()
            for d in range(ML_D):
                pltpu.make_async_copy(
                    mlu_hbm.at[olu.at[d]],
                    bmlu.at[d, pl.ds(c * CH, CH)], sem).wait()
                pltpu.make_async_copy(
                    mli_hbm.at[oli.at[d]],
                    bmli.at[d, pl.ds(c * CH, CH)], sem).wait()
            return carry

        lax.fori_loop(0, NCH, chunk, 0)

        pltpu.sync_copy(bmfu, omfu.at[:, pl.ds(base, RPW)])
        pltpu.sync_copy(bmfi, omfi.at[:, pl.ds(base, RPW)])
        pltpu.sync_copy(bmlu, omlu.at[:, pl.ds(base, RPW)])
        pltpu.sync_copy(bmli, omli.at[:, pl.ds(base, RPW)])

    return k(u1, i1, mfu_f, mfi_f, mlu_f, mli_f)


def _tc_body(mfu_ref, mfi_ref, mlu_ref, mli_ref, w1ut_ref, w1it_ref, b1_ref,
             w2t_ref, b2_ref, w3t_ref, b3_ref, wpmf_ref, wpml_ref, bp_ref,
             out_ref):
    h = jnp.dot(w1ut_ref[...], mlu_ref[...], preferred_element_type=jnp.float32)
    h = h + jnp.dot(w1it_ref[...], mli_ref[...],
                    preferred_element_type=jnp.float32)
    h = jnp.maximum(h + b1_ref[...], 0.0)
    h = jnp.maximum(
        jnp.dot(w2t_ref[...], h, preferred_element_type=jnp.float32)
        + b2_ref[...], 0.0)
    h = jnp.maximum(
        jnp.dot(w3t_ref[...], h, preferred_element_type=jnp.float32)
        + b3_ref[...], 0.0)
    mf = mfu_ref[...] * mfi_ref[...]
    logit = (jnp.dot(wpmf_ref[...], mf, preferred_element_type=jnp.float32)
             + jnp.dot(wpml_ref[...], h, preferred_element_type=jnp.float32)
             + bp_ref[...])
    out_ref[...] = jax.nn.sigmoid(logit)


def _tc_dense(mfu, mfi, mlu, mli, w1ut, w1it, b1, w2t, b2, w3t, b3,
              wp_mf, wp_ml, bp):
    BB = 2048
    grid = (BATCH // BB,)
    full = lambda shape: pl.BlockSpec(shape, lambda n: (0, 0))
    return pl.pallas_call(
        _tc_body,
        grid=grid,
        in_specs=[
            pl.BlockSpec((MF_D, BB), lambda n: (0, n)),
            pl.BlockSpec((MF_D, BB), lambda n: (0, n)),
            pl.BlockSpec((ML_D, BB), lambda n: (0, n)),
            pl.BlockSpec((ML_D, BB), lambda n: (0, n)),
            full((32, ML_D)),
            full((32, ML_D)),
            full((32, 1)),
            full((16, 32)),
            full((16, 1)),
            full((8, 16)),
            full((8, 1)),
            full((1, MF_D)),
            full((1, 8)),
            full((1, 1)),
        ],
        out_specs=pl.BlockSpec((1, BB), lambda n: (0, n)),
        out_shape=jax.ShapeDtypeStruct((1, BATCH), jnp.float32),
    )(mfu, mfi, mlu, mli, w1ut, w1it, b1, w2t, b2, w3t, b3, wp_mf, wp_ml, bp)


def kernel(user_input, item_input, mf_user, mf_item, mlp_user, mlp_item,
           W1, b1, W2, b2, W3, b3, Wp, bp):
    u1 = user_input.reshape(BATCH)
    i1 = item_input.reshape(BATCH)
    mfu, mfi, mlu, mli = _sc_gather(
        u1, i1, mf_user.T.reshape(-1), mf_item.T.reshape(-1),
        mlp_user.T.reshape(-1), mlp_item.T.reshape(-1))
    out_t = _tc_dense(
        mfu, mfi, mlu, mli,
        W1[:ML_D].T, W1[ML_D:].T, b1.reshape(-1, 1),
        W2.T, b2.reshape(-1, 1), W3.T, b3.reshape(-1, 1),
        Wp[:MF_D].T, Wp[MF_D:].T, bp.reshape(1, 1))
    return out_t.reshape(BATCH, 1)


# speedup vs baseline: 2.6266x; 2.6266x over previous
import functools

import jax
import jax.numpy as jnp
from jax import lax
from jax.experimental import pallas as pl
from jax.experimental.pallas import tpu as pltpu
from jax.experimental.pallas import tpu_sc as plsc

NC = 2
NS = 16
NW = NC * NS
BATCH = 16384
CHUNK = 128
NCHT = BATCH // CHUNK
NCH = NCHT // NW
MF_D = 10
ML_D = 32


def _sc_gather(u2d, i2d, mf_user, mf_item, mlp_user, mlp_item):
    mesh = plsc.VectorSubcoreMesh(core_axis_name="c", subcore_axis_name="s")

    @functools.partial(
        pl.kernel,
        mesh=mesh,
        compiler_params=pltpu.CompilerParams(use_tc_tiling_on_sc=False),
        out_type=(
            jax.ShapeDtypeStruct((NCHT, CHUNK, MF_D), jnp.float32),
            jax.ShapeDtypeStruct((NCHT, CHUNK, MF_D), jnp.float32),
            jax.ShapeDtypeStruct((NCHT, CHUNK, ML_D), jnp.float32),
            jax.ShapeDtypeStruct((NCHT, CHUNK, ML_D), jnp.float32),
        ),
        scratch_types=[
            pltpu.VMEM((NCH, CHUNK), jnp.int32),
            pltpu.VMEM((NCH, CHUNK), jnp.int32),
            pltpu.VMEM((NCH, CHUNK, MF_D), jnp.float32),
            pltpu.VMEM((NCH, CHUNK, MF_D), jnp.float32),
            pltpu.VMEM((NCH, CHUNK, ML_D), jnp.float32),
            pltpu.VMEM((NCH, CHUNK, ML_D), jnp.float32),
            pltpu.SemaphoreType.DMA,
        ],
    )
    def k(u_hbm, i_hbm, mfu_hbm, mfi_hbm, mlu_hbm, mli_hbm,
          mfu_out, mfi_out, mlu_out, mli_out,
          idx_u, idx_i, b_mfu, b_mfi, b_mlu, b_mli, sem):
        w = lax.axis_index("s") * NC + lax.axis_index("c")
        r0 = w * NCH
        pltpu.sync_copy(u_hbm.at[pl.ds(r0, NCH)], idx_u)
        pltpu.sync_copy(i_hbm.at[pl.ds(r0, NCH)], idx_i)
        cps = []
        for c in range(NCH):
            cps.append(pltpu.async_copy(mfu_hbm.at[idx_u.at[c]], b_mfu.at[c], sem))
            cps.append(pltpu.async_copy(mfi_hbm.at[idx_i.at[c]], b_mfi.at[c], sem))
            cps.append(pltpu.async_copy(mlu_hbm.at[idx_u.at[c]], b_mlu.at[c], sem))
            cps.append(pltpu.async_copy(mli_hbm.at[idx_i.at[c]], b_mli.at[c], sem))
        for cp in cps:
            cp.wait()
        pltpu.sync_copy(b_mfu, mfu_out.at[pl.ds(r0, NCH)])
        pltpu.sync_copy(b_mfi, mfi_out.at[pl.ds(r0, NCH)])
        pltpu.sync_copy(b_mlu, mlu_out.at[pl.ds(r0, NCH)])
        pltpu.sync_copy(b_mli, mli_out.at[pl.ds(r0, NCH)])

    return k(u2d, i2d, mf_user, mf_item, mlp_user, mlp_item)


def _tc_body(mfu_ref, mfi_ref, mlu_ref, mli_ref, w1u_ref, w1i_ref, b1_ref,
             w2_ref, b2_ref, w3_ref, b3_ref, wpmf_ref, wpml_ref, bp_ref,
             out_ref):
    h = jnp.dot(mlu_ref[...], w1u_ref[...], preferred_element_type=jnp.float32)
    h = h + jnp.dot(mli_ref[...], w1i_ref[...], preferred_element_type=jnp.float32)
    h = jnp.maximum(h + b1_ref[...], 0.0)
    h = jnp.maximum(
        jnp.dot(h, w2_ref[...], preferred_element_type=jnp.float32) + b2_ref[...], 0.0)
    h = jnp.maximum(
        jnp.dot(h, w3_ref[...], preferred_element_type=jnp.float32) + b3_ref[...], 0.0)
    mf = mfu_ref[...] * mfi_ref[...]
    logit = (jnp.dot(mf, wpmf_ref[...], preferred_element_type=jnp.float32)
             + jnp.dot(h, wpml_ref[...], preferred_element_type=jnp.float32)
             + bp_ref[...])
    out_ref[...] = jax.nn.sigmoid(logit)


def _tc_dense(mfu, mfi, mlu, mli, w1u, w1i, b1, W2, b2, W3, b3,
              wp_mf, wp_ml, bp):
    BB = 2048
    grid = (BATCH // BB,)
    full = lambda shape: pl.BlockSpec(shape, lambda n: (0, 0))
    return pl.pallas_call(
        _tc_body,
        grid=grid,
        in_specs=[
            pl.BlockSpec((BB, MF_D), lambda n: (n, 0)),
            pl.BlockSpec((BB, MF_D), lambda n: (n, 0)),
            pl.BlockSpec((BB, ML_D), lambda n: (n, 0)),
            pl.BlockSpec((BB, ML_D), lambda n: (n, 0)),
            full((ML_D, 32)),
            full((ML_D, 32)),
            full((1, 32)),
            full((32, 16)),
            full((1, 16)),
            full((16, 8)),
            full((1, 8)),
            full((MF_D, 1)),
            full((8, 1)),
            full((1, 1)),
        ],
        out_specs=pl.BlockSpec((BB, 1), lambda n: (n, 0)),
        out_shape=jax.ShapeDtypeStruct((BATCH, 1), jnp.float32),
    )(mfu, mfi, mlu, mli, w1u, w1i, b1, W2, b2, W3, b3, wp_mf, wp_ml, bp)


def kernel(user_input, item_input, mf_user, mf_item, mlp_user, mlp_item,
           W1, b1, W2, b2, W3, b3, Wp, bp):
    u2d = user_input.reshape(NCHT, CHUNK)
    i2d = item_input.reshape(NCHT, CHUNK)
    mfu, mfi, mlu, mli = _sc_gather(u2d, i2d, mf_user, mf_item,
                                    mlp_user, mlp_item)
    mfu = mfu.reshape(BATCH, MF_D)
    mfi = mfi.reshape(BATCH, MF_D)
    mlu = mlu.reshape(BATCH, ML_D)
    mli = mli.reshape(BATCH, ML_D)
    w1u = W1[:ML_D]
    w1i = W1[ML_D:]
    wp_mf = Wp[:MF_D]
    wp_ml = Wp[MF_D:]
    return _tc_dense(mfu, mfi, mlu, mli, w1u, w1i, b1.reshape(1, -1),
                     W2, b2.reshape(1, -1), W3, b3.reshape(1, -1),
                     wp_mf, wp_ml, bp.reshape(1, 1))
